# baseline (speedup 1.0000x reference)
import functools

import jax
import jax.numpy as jnp
from jax import lax
from jax.experimental import pallas as pl
from jax.experimental.pallas import tpu as pltpu
from jax.experimental.pallas import tpu_sc as plsc

_INTERPRET = False

N_EMBD = 768
N_EXPERTS = 8
TOP_K = 2
D_FF = 4 * N_EMBD

N_TOKENS = 4096
N_PAIRS = N_TOKENS * TOP_K
BLK = 512
NBLK = N_PAIRS // BLK + N_EXPERTS
CAP = NBLK * BLK

SC_CORES = 2
SC_SUBCORES = 16
SC_WORKERS = SC_CORES * SC_SUBCORES
GATHER_CHUNK = 64


def _gate_body(x_ref, gw_ref, gb_ref, idx_ref, w_ref, rank_ref, cnt_ref,
               carry_ref):
    i = pl.program_id(0)

    @pl.when(i == 0)
    def _():
        carry_ref[...] = jnp.zeros_like(carry_ref)

    logits = jnp.dot(x_ref[...], gw_ref[...],
                     preferred_element_type=jnp.float32) + gb_ref[...]
    m = jnp.max(logits, axis=-1, keepdims=True)
    e = jnp.exp(logits - m)
    iota8 = lax.broadcasted_iota(jnp.int32, e.shape, 1)
    e1 = jnp.max(e, axis=-1, keepdims=True)
    i1 = jnp.min(jnp.where(e == e1, iota8, N_EXPERTS), axis=-1, keepdims=True)
    masked = jnp.where(iota8 == i1, -1.0, e)
    e2 = jnp.max(masked, axis=-1, keepdims=True)
    i2 = jnp.min(jnp.where(masked == e2, iota8, N_EXPERTS), axis=-1,
                 keepdims=True)
    denom = e1 + e2
    idx_ref[...] = jnp.concatenate([i1, i2], axis=1)
    w_ref[...] = jnp.concatenate([e1 / denom, e2 / denom], axis=1)

    tb = e.shape[0]
    oh1 = (iota8 == i1).astype(jnp.float32)
    oh2 = (iota8 == i2).astype(jnp.float32)
    r = lax.broadcasted_iota(jnp.int32, (tb, tb), 0)
    c = lax.broadcasted_iota(jnp.int32, (tb, tb), 1)
    tri = (c < r).astype(jnp.float32)
    ex1 = jnp.dot(tri, oh1, preferred_element_type=jnp.float32)
    ex2 = jnp.dot(tri, oh2, preferred_element_type=jnp.float32)
    carry = carry_ref[...]
    cs1 = jnp.sum(oh1, axis=0, keepdims=True)
    cs2 = jnp.sum(oh2, axis=0, keepdims=True)
    rank1 = jnp.sum((carry + ex1) * oh1, axis=1, keepdims=True)
    rank2 = jnp.sum((carry + cs1 + ex2) * oh2, axis=1, keepdims=True)
    rank_ref[...] = jnp.concatenate([rank1, rank2], axis=1).astype(jnp.int32)
    new_carry = carry + cs1 + cs2
    carry_ref[...] = new_carry
    cnt_ref[...] = new_carry.astype(jnp.int32)


def _gate(xr, gate_W, gate_b):
    tb = 1024
    return pl.pallas_call(
        _gate_body,
        grid=(N_TOKENS // tb,),
        in_specs=[
            pl.BlockSpec((tb, N_EMBD), lambda i: (i, 0)),
            pl.BlockSpec((N_EMBD, N_EXPERTS), lambda i: (0, 0)),
            pl.BlockSpec((1, N_EXPERTS), lambda i: (0, 0)),
        ],
        out_specs=[
            pl.BlockSpec((tb, TOP_K), lambda i: (i, 0)),
            pl.BlockSpec((tb, TOP_K), lambda i: (i, 0)),
            pl.BlockSpec((tb, TOP_K), lambda i: (i, 0)),
            pl.BlockSpec((1, N_EXPERTS), lambda i: (0, 0)),
        ],
        out_shape=[
            jax.ShapeDtypeStruct((N_TOKENS, TOP_K), jnp.int32),
            jax.ShapeDtypeStruct((N_TOKENS, TOP_K), jnp.float32),
            jax.ShapeDtypeStruct((N_TOKENS, TOP_K), jnp.int32),
            jax.ShapeDtypeStruct((1, N_EXPERTS), jnp.int32),
        ],
        scratch_shapes=[pltpu.VMEM((1, N_EXPERTS), jnp.float32)],
        interpret=_INTERPRET,
    )(xr, gate_W, gate_b.reshape(1, N_EXPERTS))


def _sc_dispatch(xr, pair_token, dest):
    per_w = N_PAIRS // SC_WORKERS
    chunk = GATHER_CHUNK
    mesh = plsc.VectorSubcoreMesh(core_axis_name="c", subcore_axis_name="s")

    @functools.partial(
        pl.kernel, mesh=mesh,
        out_type=jax.ShapeDtypeStruct((CAP, N_EMBD), jnp.float32),
        scratch_types=[
            pltpu.VMEM((chunk,), jnp.int32),
            pltpu.VMEM((chunk,), jnp.int32),
            pltpu.VMEM((chunk, N_EMBD), jnp.float32),
            pltpu.SemaphoreType.DMA,
        ],
    )
    def dispatch_kernel(xr_hbm, tok_hbm, dest_hbm, out_hbm, tok_v, dest_v,
                        rows_v, sem):
        wid = lax.axis_index("s") * SC_CORES + lax.axis_index("c")
        base = wid * per_w

        @pl.loop(0, per_w, step=chunk)
        def _(off):
            pltpu.sync_copy(tok_hbm.at[pl.ds(base + off, chunk)], tok_v)
            pltpu.sync_copy(dest_hbm.at[pl.ds(base + off, chunk)], dest_v)
            pltpu.async_copy(xr_hbm.at[tok_v], rows_v, sem).wait()
            pltpu.sync_copy(rows_v, out_hbm.at[dest_v])

    return dispatch_kernel(xr, pair_token, dest)


def _sc_gather(table, idx, n_rows):
    d = table.shape[1]
    per_w = n_rows // SC_WORKERS
    chunk = min(GATHER_CHUNK, per_w)
    mesh = plsc.VectorSubcoreMesh(core_axis_name="c", subcore_axis_name="s")

    @functools.partial(
        pl.kernel, mesh=mesh,
        out_type=jax.ShapeDtypeStruct((n_rows, d), table.dtype),
        scratch_types=[
            pltpu.VMEM((chunk,), jnp.int32),
            pltpu.VMEM((chunk, d), table.dtype),
            pltpu.SemaphoreType.DMA,
        ],
    )
    def gather_kernel(table_hbm, idx_hbm, out_hbm, idx_v, rows_v, sem):
        wid = lax.axis_index("s") * SC_CORES + lax.axis_index("c")
        base = wid * per_w

        @pl.loop(0, per_w, step=chunk)
        def _(off):
            pltpu.sync_copy(idx_hbm.at[pl.ds(base + off, chunk)], idx_v)
            pltpu.async_copy(table_hbm.at[idx_v], rows_v, sem).wait()
            pltpu.sync_copy(rows_v, out_hbm.at[pl.ds(base + off, chunk)])

    return gather_kernel(table, idx)


DFF_SPLIT = 2
DFF_CHUNK = D_FF // DFF_SPLIT


def _ffn_body(be_ref, x_ref, W1_ref, b1_ref, W2_ref, b2_ref, o_ref, acc_ref):
    j = pl.program_id(1)
    h = jnp.dot(x_ref[...], W1_ref[0], preferred_element_type=jnp.float32)
    h = jnp.maximum(h + b1_ref[0], 0.0)
    y = jnp.dot(h, W2_ref[0], preferred_element_type=jnp.float32)

    @pl.when(j == 0)
    def _():
        acc_ref[...] = y + b2_ref[0]

    @pl.when(j == DFF_SPLIT - 1)
    def _():
        o_ref[...] = acc_ref[...] + y


def _ffn(block_expert, x_sorted, W1, b1, W2, b2):
    grid_spec = pltpu.PrefetchScalarGridSpec(
        num_scalar_prefetch=1,
        grid=(NBLK, DFF_SPLIT),
        in_specs=[
            pl.BlockSpec((BLK, N_EMBD), lambda i, j, be: (i, 0)),
            pl.BlockSpec((1, N_EMBD, DFF_CHUNK), lambda i, j, be: (be[i], 0, j)),
            pl.BlockSpec((1, 1, DFF_CHUNK), lambda i, j, be: (be[i], 0, j)),
            pl.BlockSpec((1, DFF_CHUNK, N_EMBD), lambda i, j, be: (be[i], j, 0)),
            pl.BlockSpec((1, 1, N_EMBD), lambda i, j, be: (be[i], 0, 0)),
        ],
        out_specs=pl.BlockSpec((BLK, N_EMBD), lambda i, j, be: (i, 0)),
        scratch_shapes=[pltpu.VMEM((BLK, N_EMBD), jnp.float32)],
    )
    return pl.pallas_call(
        _ffn_body,
        grid_spec=grid_spec,
        out_shape=jax.ShapeDtypeStruct((CAP, N_EMBD), jnp.float32),
        interpret=_INTERPRET,
    )(block_expert, x_sorted,
      W1, b1.reshape(N_EXPERTS, 1, D_FF), W2, b2.reshape(N_EXPERTS, 1, N_EMBD))


def _combine_body(w_ref, a_ref, o_ref):
    o_ref[...] = (w_ref[:, :1] * a_ref[:, :N_EMBD]
                  + w_ref[:, 1:] * a_ref[:, N_EMBD:])


def _combine(w, y_pairs2):
    tb = 512
    return pl.pallas_call(
        _combine_body,
        grid=(N_TOKENS // tb,),
        in_specs=[
            pl.BlockSpec((tb, TOP_K), lambda i: (i, 0)),
            pl.BlockSpec((tb, 2 * N_EMBD), lambda i: (i, 0)),
        ],
        out_specs=pl.BlockSpec((tb, N_EMBD), lambda i: (i, 0)),
        out_shape=jax.ShapeDtypeStruct((N_TOKENS, N_EMBD), jnp.float32),
        compiler_params=pltpu.CompilerParams(
            dimension_semantics=("parallel",)),
        interpret=_INTERPRET,
    )(w, y_pairs2)


def kernel(x, gate_W, gate_b, W1, b1, W2, b2):
    B, T, C = x.shape
    xr = x.reshape(N_TOKENS, C)

    idx, w, rank, cnt = _gate(xr, gate_W, gate_b)

    counts = cnt[0]
    bpe = (counts + BLK - 1) // BLK
    blk_start = jnp.concatenate([jnp.zeros((1,), jnp.int32),
                                 jnp.cumsum(bpe).astype(jnp.int32)])
    padded_off = blk_start[:N_EXPERTS] * BLK
    flat_e = idx.reshape(N_PAIRS)
    oh = (flat_e[:, None] == jnp.arange(N_EXPERTS)[None, :])
    dest = (jnp.sum(jnp.where(oh, padded_off[None, :], 0), axis=1)
            + rank.reshape(N_PAIRS)).astype(jnp.int32)
    block_expert = jnp.clip(
        jnp.sum(jnp.arange(NBLK, dtype=jnp.int32)[:, None]
                >= blk_start[1:][None, :], axis=1),
        0, N_EXPERTS - 1).astype(jnp.int32)
    pair_token = jnp.arange(N_PAIRS, dtype=jnp.int32) // TOP_K

    x_sorted = _sc_dispatch(xr, pair_token, dest)
    y_sorted = _ffn(block_expert, x_sorted, W1, b1, W2, b2)
    y_pairs = _sc_gather(y_sorted, dest, N_PAIRS)
    out = _combine(w, y_pairs.reshape(N_TOKENS, 2 * N_EMBD))
    return out.reshape(B, T, C)


# speedup vs baseline: 1.1204x; 1.1204x over previous
import functools

import jax
import jax.numpy as jnp
from jax import lax
from jax.experimental import pallas as pl
from jax.experimental.pallas import tpu as pltpu
from jax.experimental.pallas import tpu_sc as plsc

_INTERPRET = False

N_EMBD = 768
N_EXPERTS = 8
TOP_K = 2
D_FF = 4 * N_EMBD

N_TOKENS = 4096
N_PAIRS = N_TOKENS * TOP_K
BLK = 512
NBLK = N_PAIRS // BLK + N_EXPERTS
CAP = NBLK * BLK

SC_CORES = 2
SC_SUBCORES = 16
SC_WORKERS = SC_CORES * SC_SUBCORES
GATHER_CHUNK = 128


def _gate_body(x_ref, gw_ref, gb_ref, idx_ref, w_ref, rank_ref, cnt_ref,
               carry_ref):
    i = pl.program_id(0)

    @pl.when(i == 0)
    def _():
        carry_ref[...] = jnp.zeros_like(carry_ref)

    logits = jnp.dot(x_ref[...], gw_ref[...],
                     preferred_element_type=jnp.float32) + gb_ref[...]
    m = jnp.max(logits, axis=-1, keepdims=True)
    e = jnp.exp(logits - m)
    iota8 = lax.broadcasted_iota(jnp.int32, e.shape, 1)
    e1 = jnp.max(e, axis=-1, keepdims=True)
    i1 = jnp.min(jnp.where(e == e1, iota8, N_EXPERTS), axis=-1, keepdims=True)
    masked = jnp.where(iota8 == i1, -1.0, e)
    e2 = jnp.max(masked, axis=-1, keepdims=True)
    i2 = jnp.min(jnp.where(masked == e2, iota8, N_EXPERTS), axis=-1,
                 keepdims=True)
    denom = e1 + e2
    idx_ref[...] = jnp.concatenate([i1, i2], axis=1)
    w_ref[...] = jnp.concatenate([e1 / denom, e2 / denom], axis=1)

    tb = e.shape[0]
    oh1 = (iota8 == i1).astype(jnp.float32)
    oh2 = (iota8 == i2).astype(jnp.float32)
    r = lax.broadcasted_iota(jnp.int32, (tb, tb), 0)
    c = lax.broadcasted_iota(jnp.int32, (tb, tb), 1)
    tri = (c < r).astype(jnp.float32)
    ex1 = jnp.dot(tri, oh1, preferred_element_type=jnp.float32)
    ex2 = jnp.dot(tri, oh2, preferred_element_type=jnp.float32)
    carry = carry_ref[...]
    cs1 = jnp.sum(oh1, axis=0, keepdims=True)
    cs2 = jnp.sum(oh2, axis=0, keepdims=True)
    rank1 = jnp.sum((carry + ex1) * oh1, axis=1, keepdims=True)
    rank2 = jnp.sum((carry + cs1 + ex2) * oh2, axis=1, keepdims=True)
    rank_ref[...] = jnp.concatenate([rank1, rank2], axis=1).astype(jnp.int32)
    new_carry = carry + cs1 + cs2
    carry_ref[...] = new_carry
    cnt_ref[...] = new_carry.astype(jnp.int32)


def _gate(xr, gate_W, gate_b):
    tb = 1024
    return pl.pallas_call(
        _gate_body,
        grid=(N_TOKENS // tb,),
        in_specs=[
            pl.BlockSpec((tb, N_EMBD), lambda i: (i, 0)),
            pl.BlockSpec((N_EMBD, N_EXPERTS), lambda i: (0, 0)),
            pl.BlockSpec((1, N_EXPERTS), lambda i: (0, 0)),
        ],
        out_specs=[
            pl.BlockSpec((tb, TOP_K), lambda i: (i, 0)),
            pl.BlockSpec((tb, TOP_K), lambda i: (i, 0)),
            pl.BlockSpec((tb, TOP_K), lambda i: (i, 0)),
            pl.BlockSpec((1, N_EXPERTS), lambda i: (0, 0)),
        ],
        out_shape=[
            jax.ShapeDtypeStruct((N_TOKENS, TOP_K), jnp.int32),
            jax.ShapeDtypeStruct((N_TOKENS, TOP_K), jnp.float32),
            jax.ShapeDtypeStruct((N_TOKENS, TOP_K), jnp.int32),
            jax.ShapeDtypeStruct((1, N_EXPERTS), jnp.int32),
        ],
        scratch_shapes=[pltpu.VMEM((1, N_EXPERTS), jnp.float32)],
        interpret=_INTERPRET,
    )(xr, gate_W, gate_b.reshape(1, N_EXPERTS))


def _sc_dispatch(xr, pair_token, dest):
    per_w = N_PAIRS // SC_WORKERS
    chunk = GATHER_CHUNK
    mesh = plsc.VectorSubcoreMesh(core_axis_name="c", subcore_axis_name="s")

    @functools.partial(
        pl.kernel, mesh=mesh,
        out_type=jax.ShapeDtypeStruct((CAP, N_EMBD), jnp.float32),
        scratch_types=[
            pltpu.VMEM((chunk,), jnp.int32),
            pltpu.VMEM((chunk,), jnp.int32),
            pltpu.VMEM((chunk, N_EMBD), jnp.float32),
            pltpu.SemaphoreType.DMA,
        ],
    )
    def dispatch_kernel(xr_hbm, tok_hbm, dest_hbm, out_hbm, tok_v, dest_v,
                        rows_v, sem):
        wid = lax.axis_index("s") * SC_CORES + lax.axis_index("c")
        base = wid * per_w

        @pl.loop(0, per_w, step=chunk)
        def _(off):
            pltpu.sync_copy(tok_hbm.at[pl.ds(base + off, chunk)], tok_v)
            pltpu.sync_copy(dest_hbm.at[pl.ds(base + off, chunk)], dest_v)
            pltpu.async_copy(xr_hbm.at[tok_v], rows_v, sem).wait()
            pltpu.sync_copy(rows_v, out_hbm.at[dest_v])

    return dispatch_kernel(xr, pair_token, dest)


def _sc_gather(table, idx, n_rows):
    d = table.shape[1]
    per_w = n_rows // SC_WORKERS
    chunk = min(GATHER_CHUNK, per_w)
    mesh = plsc.VectorSubcoreMesh(core_axis_name="c", subcore_axis_name="s")

    @functools.partial(
        pl.kernel, mesh=mesh,
        out_type=jax.ShapeDtypeStruct((n_rows, d), table.dtype),
        scratch_types=[
            pltpu.VMEM((chunk,), jnp.int32),
            pltpu.VMEM((chunk, d), table.dtype),
            pltpu.SemaphoreType.DMA,
        ],
    )
    def gather_kernel(table_hbm, idx_hbm, out_hbm, idx_v, rows_v, sem):
        wid = lax.axis_index("s") * SC_CORES + lax.axis_index("c")
        base = wid * per_w

        @pl.loop(0, per_w, step=chunk)
        def _(off):
            pltpu.sync_copy(idx_hbm.at[pl.ds(base + off, chunk)], idx_v)
            pltpu.async_copy(table_hbm.at[idx_v], rows_v, sem).wait()
            pltpu.sync_copy(rows_v, out_hbm.at[pl.ds(base + off, chunk)])

    return gather_kernel(table, idx)


def _ffn_body(be_ref, x_ref, W1_ref, b1_ref, W2_ref, b2_ref, o_ref):
    h = jnp.dot(x_ref[...], W1_ref[0], preferred_element_type=jnp.float32)
    h = jnp.maximum(h + b1_ref[0], 0.0)
    y = jnp.dot(h, W2_ref[0], preferred_element_type=jnp.float32)
    o_ref[...] = y + b2_ref[0]


def _ffn(block_expert, x_sorted, W1, b1, W2, b2):
    grid_spec = pltpu.PrefetchScalarGridSpec(
        num_scalar_prefetch=1,
        grid=(NBLK,),
        in_specs=[
            pl.BlockSpec((BLK, N_EMBD), lambda i, be: (i, 0)),
            pl.BlockSpec((1, N_EMBD, D_FF), lambda i, be: (be[i], 0, 0)),
            pl.BlockSpec((1, 1, D_FF), lambda i, be: (be[i], 0, 0)),
            pl.BlockSpec((1, D_FF, N_EMBD), lambda i, be: (be[i], 0, 0)),
            pl.BlockSpec((1, 1, N_EMBD), lambda i, be: (be[i], 0, 0)),
        ],
        out_specs=pl.BlockSpec((BLK, N_EMBD), lambda i, be: (i, 0)),
    )
    return pl.pallas_call(
        _ffn_body,
        grid_spec=grid_spec,
        out_shape=jax.ShapeDtypeStruct((CAP, N_EMBD), jnp.float32),
        compiler_params=pltpu.CompilerParams(
            dimension_semantics=("parallel",)),
        interpret=_INTERPRET,
    )(block_expert, x_sorted,
      W1, b1.reshape(N_EXPERTS, 1, D_FF), W2, b2.reshape(N_EXPERTS, 1, N_EMBD))


def _combine_body(w_ref, a_ref, o_ref):
    o_ref[...] = (w_ref[:, :1] * a_ref[:, :N_EMBD]
                  + w_ref[:, 1:] * a_ref[:, N_EMBD:])


def _combine(w, y_pairs2):
    tb = 512
    return pl.pallas_call(
        _combine_body,
        grid=(N_TOKENS // tb,),
        in_specs=[
            pl.BlockSpec((tb, TOP_K), lambda i: (i, 0)),
            pl.BlockSpec((tb, 2 * N_EMBD), lambda i: (i, 0)),
        ],
        out_specs=pl.BlockSpec((tb, N_EMBD), lambda i: (i, 0)),
        out_shape=jax.ShapeDtypeStruct((N_TOKENS, N_EMBD), jnp.float32),
        compiler_params=pltpu.CompilerParams(
            dimension_semantics=("parallel",)),
        interpret=_INTERPRET,
    )(w, y_pairs2)


def kernel(x, gate_W, gate_b, W1, b1, W2, b2):
    B, T, C = x.shape
    xr = x.reshape(N_TOKENS, C)

    idx, w, rank, cnt = _gate(xr, gate_W, gate_b)

    counts = cnt[0]
    bpe = (counts + BLK - 1) // BLK
    blk_start = jnp.concatenate([jnp.zeros((1,), jnp.int32),
                                 jnp.cumsum(bpe).astype(jnp.int32)])
    padded_off = blk_start[:N_EXPERTS] * BLK
    flat_e = idx.reshape(N_PAIRS)
    oh = (flat_e[:, None] == jnp.arange(N_EXPERTS)[None, :])
    dest = (jnp.sum(jnp.where(oh, padded_off[None, :], 0), axis=1)
            + rank.reshape(N_PAIRS)).astype(jnp.int32)
    block_expert = jnp.clip(
        jnp.sum(jnp.arange(NBLK, dtype=jnp.int32)[:, None]
                >= blk_start[1:][None, :], axis=1),
        0, N_EXPERTS - 1).astype(jnp.int32)
    pair_token = jnp.arange(N_PAIRS, dtype=jnp.int32) // TOP_K

    x_sorted = _sc_dispatch(xr, pair_token, dest)
    y_sorted = _ffn(block_expert, x_sorted, W1, b1, W2, b2)
    y_pairs = _sc_gather(y_sorted, dest, N_PAIRS)
    out = _combine(w, y_pairs.reshape(N_TOKENS, 2 * N_EMBD))
    return out.reshape(B, T, C)
